# baseline (speedup 1.0000x reference)
import functools

import jax
import jax.numpy as jnp
from jax.experimental import pallas as pl
from jax.experimental.pallas import tpu as pltpu

DIM = 2048
INTER = 1408
NEXP = 8
TOPK = 2
SHARED_INTER = 2 * INTER
T = 2048
NASN = T * TOPK
BROW = 128
PAD_N = NASN + NEXP * BROW
NBLK = PAD_N // BROW

GATE_BT = 512
SH_BT = 256
SH_IB = 256
SH_NI = SHARED_INTER // SH_IB


def _gate_body(x_ref, gw_ref, gb_ref, idx_ref, w_ref):
    xv = x_ref[...]
    logits = jax.lax.dot_general(
        xv, gw_ref[...], (((1,), (1,)), ((), ())),
        preferred_element_type=jnp.float32)
    m = jnp.max(logits, axis=1, keepdims=True)
    p = jnp.exp(logits - m)
    orig = p / jnp.sum(p, axis=1, keepdims=True)
    s2 = orig + gb_ref[...]
    lane = jax.lax.broadcasted_iota(jnp.int32, (GATE_BT, NEXP), 1)
    m1 = jnp.max(s2, axis=1, keepdims=True)
    idx1 = jnp.min(jnp.where(s2 == m1, lane, NEXP), axis=1, keepdims=True)
    s2m = jnp.where(lane == idx1, -jnp.inf, s2)
    m2 = jnp.max(s2m, axis=1, keepdims=True)
    idx2 = jnp.min(jnp.where(s2m == m2, lane, NEXP), axis=1, keepdims=True)
    w1 = jnp.sum(jnp.where(lane == idx1, orig, 0.0), axis=1, keepdims=True)
    w2 = jnp.sum(jnp.where(lane == idx2, orig, 0.0), axis=1, keepdims=True)
    idx_ref[...] = jnp.concatenate([idx1, idx2], axis=1)
    w_ref[...] = jnp.concatenate([w1, w2], axis=1)


def _gate(xt, gate_w, gate_b):
    return pl.pallas_call(
        _gate_body,
        grid=(T // GATE_BT,),
        in_specs=[
            pl.BlockSpec((GATE_BT, DIM), lambda t: (t, 0)),
            pl.BlockSpec((NEXP, DIM), lambda t: (0, 0)),
            pl.BlockSpec((1, NEXP), lambda t: (0, 0)),
        ],
        out_specs=[
            pl.BlockSpec((GATE_BT, TOPK), lambda t: (t, 0)),
            pl.BlockSpec((GATE_BT, TOPK), lambda t: (t, 0)),
        ],
        out_shape=[
            jax.ShapeDtypeStruct((T, TOPK), jnp.int32),
            jax.ShapeDtypeStruct((T, TOPK), jnp.float32),
        ],
    )(xt, gate_w, gate_b.reshape(1, NEXP))


def _gemm_h_body(be_ref, x_ref, w1_ref, w3_ref, h_ref):
    xv = x_ref[...]
    h1 = jax.lax.dot_general(xv, w1_ref[0], (((1,), (1,)), ((), ())),
                             preferred_element_type=jnp.float32)
    h3 = jax.lax.dot_general(xv, w3_ref[0], (((1,), (1,)), ((), ())),
                             preferred_element_type=jnp.float32)
    h_ref[...] = h1 * jax.nn.sigmoid(h1) * h3


def _gemm_y_body(be_ref, h_ref, w2_ref, o_ref):
    o_ref[...] = jax.lax.dot_general(h_ref[...], w2_ref[0],
                                     (((1,), (1,)), ((), ())),
                                     preferred_element_type=jnp.float32)


def _grouped_gemm(x_sorted, we1, we3, we2, block_expert):
    h_spec = pltpu.PrefetchScalarGridSpec(
        num_scalar_prefetch=1,
        grid=(NBLK,),
        in_specs=[
            pl.BlockSpec((BROW, DIM), lambda b, be: (b, 0)),
            pl.BlockSpec((1, INTER, DIM), lambda b, be: (be[b], 0, 0)),
            pl.BlockSpec((1, INTER, DIM), lambda b, be: (be[b], 0, 0)),
        ],
        out_specs=pl.BlockSpec((BROW, INTER), lambda b, be: (b, 0)),
    )
    h = pl.pallas_call(
        _gemm_h_body,
        grid_spec=h_spec,
        out_shape=jax.ShapeDtypeStruct((PAD_N, INTER), jnp.float32),
    )(block_expert, x_sorted, we1, we3)
    y_spec = pltpu.PrefetchScalarGridSpec(
        num_scalar_prefetch=1,
        grid=(NBLK,),
        in_specs=[
            pl.BlockSpec((BROW, INTER), lambda b, be: (b, 0)),
            pl.BlockSpec((1, DIM, INTER), lambda b, be: (be[b], 0, 0)),
        ],
        out_specs=pl.BlockSpec((BROW, DIM), lambda b, be: (b, 0)),
    )
    return pl.pallas_call(
        _gemm_y_body,
        grid_spec=y_spec,
        out_shape=jax.ShapeDtypeStruct((PAD_N, DIM), jnp.float32),
    )(block_expert, h, we2)


def _shared_body(x_ref, w1_ref, w3_ref, w2_ref, o_ref, acc_ref):
    i = pl.program_id(0)
    t = pl.program_id(1)
    xv = x_ref[...]
    h1 = jax.lax.dot_general(xv, w1_ref[...], (((1,), (1,)), ((), ())),
                             preferred_element_type=jnp.float32)
    h3 = jax.lax.dot_general(xv, w3_ref[...], (((1,), (1,)), ((), ())),
                             preferred_element_type=jnp.float32)
    h = h1 * jax.nn.sigmoid(h1) * h3
    part = jax.lax.dot_general(h, w2_ref[...], (((1,), (1,)), ((), ())),
                               preferred_element_type=jnp.float32)
    rows = pl.ds(t * SH_BT, SH_BT)

    @pl.when(i == 0)
    def _():
        acc_ref[rows, :] = part

    @pl.when(i > 0)
    def _():
        acc_ref[rows, :] += part

    @pl.when(i == SH_NI - 1)
    def _():
        o_ref[...] = acc_ref[rows, :]


def _shared(xt, sw1, sw3, sw2):
    return pl.pallas_call(
        _shared_body,
        grid=(SH_NI, T // SH_BT),
        in_specs=[
            pl.BlockSpec((SH_BT, DIM), lambda i, t: (t, 0)),
            pl.BlockSpec((SH_IB, DIM), lambda i, t: (i, 0)),
            pl.BlockSpec((SH_IB, DIM), lambda i, t: (i, 0)),
            pl.BlockSpec((DIM, SH_IB), lambda i, t: (0, i)),
        ],
        out_specs=pl.BlockSpec((SH_BT, DIM), lambda i, t: (t, 0)),
        out_shape=jax.ShapeDtypeStruct((T, DIM), jnp.float32),
        scratch_shapes=[pltpu.VMEM((T, DIM), jnp.float32)],
    )(xt, sw1, sw3, sw2)


def kernel(x, gate_w, gate_b, we1, we2, we3, sw1, sw2, sw3):
    xt = x.reshape(T, DIM)
    idx, w = _gate(xt, gate_w, gate_b)

    eflat = idx.reshape(-1)
    ids = jnp.arange(NEXP, dtype=jnp.int32)
    oh = (eflat[:, None] == ids[None, :]).astype(jnp.int32)
    rank = jnp.sum((jnp.cumsum(oh, axis=0) - 1) * oh, axis=1)
    counts = jnp.sum(oh, axis=0)
    padded = ((counts + BROW - 1) // BROW) * BROW
    ends = jnp.cumsum(padded)
    offs = ends - padded
    pos = offs[eflat] + rank
    iall = jnp.arange(NASN, dtype=jnp.int32)
    sorted_token = jnp.zeros((PAD_N,), jnp.int32).at[pos].set(iall // TOPK)
    starts = jnp.arange(NBLK, dtype=jnp.int32) * BROW
    block_expert = jnp.minimum(
        jnp.sum((starts[:, None] >= ends[None, :]).astype(jnp.int32), axis=1),
        NEXP - 1).astype(jnp.int32)

    x_sorted = jnp.take(xt, sorted_token, axis=0)
    ys = _grouped_gemm(x_sorted, we1, we3, we2, block_expert)
    z = _shared(xt, sw1, sw3, sw2)

    posr = pos.reshape(T, TOPK)
    y = (jnp.take(ys, posr[:, 0], axis=0) * w[:, 0:1]
         + jnp.take(ys, posr[:, 1], axis=0) * w[:, 1:2] + z)
    return y.reshape(x.shape)
